# baseline (speedup 1.0000x reference)
import functools

import jax
import jax.numpy as jnp
from jax import lax
from jax.experimental import pallas as pl
from jax.experimental.pallas import tpu as pltpu
from jax.experimental.pallas import tpu_sc as plsc


def _sc_call(x_flat, rt_flat, *, S, H, max_len, rows):
    NW = 32
    JW = S // NW
    CI = 8
    NCH = rows // CI
    XB = JW * H
    W = (CI + JW - 1) * H
    ROW = S * H

    mesh = plsc.VectorSubcoreMesh(core_axis_name="c", subcore_axis_name="s")

    @functools.partial(
        pl.kernel,
        mesh=mesh,
        out_type=jax.ShapeDtypeStruct((rows * S * H,), jnp.float32),
        scratch_types=[
            pltpu.VMEM((XB,), jnp.float32),
            pltpu.VMEM((W,), jnp.float32),
            pltpu.VMEM((W,), jnp.float32),
            pltpu.VMEM((CI * XB,), jnp.float32),
            pltpu.VMEM((CI * XB,), jnp.float32),
            pltpu.SemaphoreType.DMA,
            pltpu.SemaphoreType.DMA,
            pltpu.SemaphoreType.DMA,
            pltpu.SemaphoreType.DMA,
        ],
    )
    def k(x_hbm, rt_hbm, out_hbm, xbuf, rta, rtb, outa, outb,
          sla, slb, soa, sob):
        wid = lax.axis_index("s") * 2 + lax.axis_index("c")
        j0 = wid * JW

        def rt_src(c):
            start = (max_len - c * CI - (CI - 1) + j0) * H
            return rt_hbm.at[pl.ds(start, W)]

        def compute(rtbuf, outbuf):
            @plsc.parallel_loop(0, XB, 16, unroll=2)
            def _inner(b):
                xv = xbuf[pl.ds(b, 16)]
                for r in range(CI):
                    outbuf[pl.ds(r * XB + b, 16)] = (
                        xv + rtbuf[pl.ds((CI - 1 - r) * H + b, 16)]
                    )

        def out_start(outbuf, sem, c):
            for r in range(CI):
                pltpu.make_async_copy(
                    outbuf.at[pl.ds(r * XB, XB)],
                    out_hbm.at[pl.ds((c * CI + r) * ROW + j0 * H, XB)],
                    sem,
                ).start()

        def out_wait(outbuf, sem):
            pltpu.make_async_copy(
                outbuf, out_hbm.at[pl.ds(0, CI * XB)], sem
            ).wait()

        pltpu.sync_copy(x_hbm.at[pl.ds(j0 * H, XB)], xbuf)
        pltpu.make_async_copy(rt_src(1), rtb, slb).start()
        pltpu.sync_copy(rt_src(0), rta)
        compute(rta, outa)
        out_start(outa, soa, 0)
        pltpu.make_async_copy(rt_src(2), rta, sla).start()
        pltpu.make_async_copy(rt_src(1), rtb, slb).wait()
        compute(rtb, outb)
        out_start(outb, sob, 1)

        def pair(cp, carry):
            ca = 2 * cp
            cb = 2 * cp + 1
            ca_next = jnp.minimum(ca + 2, NCH - 1)
            pltpu.make_async_copy(rt_src(cb), rtb, slb).start()
            pltpu.make_async_copy(rt_src(ca), rta, sla).wait()
            out_wait(outa, soa)
            compute(rta, outa)
            out_start(outa, soa, ca)
            pltpu.make_async_copy(rt_src(ca_next), rta, sla).start()
            pltpu.make_async_copy(rt_src(cb), rtb, slb).wait()
            out_wait(outb, sob)
            compute(rtb, outb)
            out_start(outb, sob, cb)
            return carry

        lax.fori_loop(1, NCH // 2, pair, 0)

        pltpu.make_async_copy(rt_src(NCH - 1), rta, sla).wait()
        out_wait(outa, soa)
        out_wait(outb, sob)

    return k(x_flat, rt_flat)


_K = 384
_BI = 8


def _tc_body(x_ref, rt_ref, out_ref, *, seq_len, max_len, row0):
    ib = pl.program_id(0)
    xv = x_ref[0]
    for r in range(_BI):
        i = row0 + ib * _BI + r
        out_ref[r] = xv + rt_ref[pl.ds(max_len - i, seq_len), :]


def _tc_call(x, rt, *, S, H, max_len, row0):
    body = functools.partial(_tc_body, seq_len=S, max_len=max_len, row0=row0)
    return pl.pallas_call(
        body,
        grid=((S - row0) // _BI,),
        in_specs=[
            pl.BlockSpec((1, S, H), lambda ib: (0, 0, 0)),
            pl.BlockSpec(rt.shape, lambda ib: (0, 0)),
        ],
        out_specs=pl.BlockSpec(
            (_BI, S, H), lambda ib: (ib + row0 // _BI, 0, 0)
        ),
        out_shape=jax.ShapeDtypeStruct((S, S, H), jnp.float32),
    )(x, rt)


def kernel(x, rel_pos_embeddings):
    _, S, H = x.shape
    n_rows = rel_pos_embeddings.shape[0]
    max_len = (n_rows - 1) // 2
    pad = (-n_rows) % 8
    rt = jnp.pad(jnp.flip(rel_pos_embeddings, axis=0), ((0, pad), (0, 0)))
    sc_piece = _sc_call(
        x.reshape(S * H), rt.reshape(-1), S=S, H=H, max_len=max_len, rows=_K
    ).reshape(_K, S, H)
    tc_full = _tc_call(x, rt, S=S, H=H, max_len=max_len, row0=_K)
    return lax.dynamic_update_slice(tc_full, sc_piece, (0, 0, 0))


# speedup vs baseline: 1.2849x; 1.2849x over previous
import functools

import jax
import jax.numpy as jnp
from jax import lax
from jax.experimental import pallas as pl
from jax.experimental.pallas import tpu as pltpu
from jax.experimental.pallas import tpu_sc as plsc


def _sc_call(x_flat, rt_flat, *, S, H, max_len):
    NW = 32
    JW = S // NW
    CI = 8
    NCH = S // CI
    XB = JW * H
    W = (CI + JW - 1) * H
    ROW = S * H

    mesh = plsc.VectorSubcoreMesh(core_axis_name="c", subcore_axis_name="s")

    @functools.partial(
        pl.kernel,
        mesh=mesh,
        out_type=jax.ShapeDtypeStruct((S * S * H,), jnp.float32),
        scratch_types=[
            pltpu.VMEM((XB,), jnp.float32),
            pltpu.VMEM((W,), jnp.float32),
            pltpu.VMEM((W,), jnp.float32),
            pltpu.VMEM((CI * XB,), jnp.float32),
            pltpu.VMEM((CI * XB,), jnp.float32),
            pltpu.SemaphoreType.DMA,
            pltpu.SemaphoreType.DMA,
            pltpu.SemaphoreType.DMA,
            pltpu.SemaphoreType.DMA,
        ],
    )
    def k(x_hbm, rt_hbm, out_hbm, xbuf, rta, rtb, outa, outb,
          sla, slb, soa, sob):
        wid = lax.axis_index("s") * 2 + lax.axis_index("c")
        j0 = wid * JW

        def rt_src(c):
            start = (max_len - c * CI - (CI - 1) + j0) * H
            return rt_hbm.at[pl.ds(start, W)]

        def compute(rtbuf, outbuf):
            @plsc.parallel_loop(0, XB, 16, unroll=2)
            def _inner(b):
                xv = xbuf[pl.ds(b, 16)]
                for r in range(CI):
                    outbuf[pl.ds(r * XB + b, 16)] = (
                        xv + rtbuf[pl.ds((CI - 1 - r) * H + b, 16)]
                    )

        def out_start(outbuf, sem, c):
            for r in range(CI):
                pltpu.make_async_copy(
                    outbuf.at[pl.ds(r * XB, XB)],
                    out_hbm.at[pl.ds((c * CI + r) * ROW + j0 * H, XB)],
                    sem,
                ).start()

        def out_wait(outbuf, sem):
            pltpu.make_async_copy(
                outbuf, out_hbm.at[pl.ds(0, CI * XB)], sem
            ).wait()

        pltpu.sync_copy(x_hbm.at[pl.ds(j0 * H, XB)], xbuf)
        pltpu.make_async_copy(rt_src(1), rtb, slb).start()
        pltpu.sync_copy(rt_src(0), rta)
        compute(rta, outa)
        out_start(outa, soa, 0)
        pltpu.make_async_copy(rt_src(2), rta, sla).start()
        pltpu.make_async_copy(rt_src(1), rtb, slb).wait()
        compute(rtb, outb)
        out_start(outb, sob, 1)

        def pair(cp, carry):
            ca = 2 * cp
            cb = 2 * cp + 1
            ca_next = jnp.minimum(ca + 2, NCH - 1)
            pltpu.make_async_copy(rt_src(cb), rtb, slb).start()
            pltpu.make_async_copy(rt_src(ca), rta, sla).wait()
            out_wait(outa, soa)
            compute(rta, outa)
            out_start(outa, soa, ca)
            pltpu.make_async_copy(rt_src(ca_next), rta, sla).start()
            pltpu.make_async_copy(rt_src(cb), rtb, slb).wait()
            out_wait(outb, sob)
            compute(rtb, outb)
            out_start(outb, sob, cb)
            return carry

        lax.fori_loop(1, NCH // 2, pair, 0)

        pltpu.make_async_copy(rt_src(NCH - 1), rta, sla).wait()
        out_wait(outa, soa)
        out_wait(outb, sob)

    return k(x_flat, rt_flat)


def kernel(x, rel_pos_embeddings):
    _, S, H = x.shape
    n_rows = rel_pos_embeddings.shape[0]
    max_len = (n_rows - 1) // 2
    pad = (-n_rows) % 8
    rt = jnp.pad(jnp.flip(rel_pos_embeddings, axis=0), ((0, pad), (0, 0)))
    out = _sc_call(
        x.reshape(S * H), rt.reshape(-1), S=S, H=H, max_len=max_len
    )
    return out.reshape(S, S, H)
